# baseline (speedup 1.0000x reference)
import jax
import jax.numpy as jnp
from jax.experimental import pallas as pl
from jax.experimental.pallas import tpu as pltpu

_BN = 30720
_NFULL = 3
_TAIL = 7840
_NIN = 2
_NOUT = 3
_S = 4


def _dot(wt_ref, x):
    return jax.lax.dot_general(
        wt_ref[...],
        x,
        dimension_numbers=(((0,), (0,)), ((), ())),
        preferred_element_type=jnp.float32,
    )


def _body(wt_ref, ft_any, o_any, ft_v, o_v, ft_t, o_t,
          in_sem, out_sem, tin_sem, tout_sem):
    h = o_any.shape[0]
    hs = h // _S

    def in_copy(i, b):
        return pltpu.make_async_copy(
            ft_any.at[:, pl.ds(i * _BN, _BN)], ft_v.at[b], in_sem.at[b])

    def out_copies(i, b):
        return [
            pltpu.make_async_copy(
                o_v.at[b, pl.ds(s * hs, hs)],
                o_any.at[pl.ds(s * hs, hs), pl.ds(i * _BN, _BN)],
                out_sem.at[b, s])
            for s in range(_S)
        ]

    tail_in = pltpu.make_async_copy(
        ft_any.at[:, pl.ds(_NFULL * _BN, _TAIL)], ft_t, tin_sem)

    def tail_outs():
        return [
            pltpu.make_async_copy(
                o_t.at[pl.ds(s * hs, hs)],
                o_any.at[pl.ds(s * hs, hs), pl.ds(_NFULL * _BN, _TAIL)],
                tout_sem.at[s])
            for s in range(_S)
        ]

    in_copy(0, 0).start()
    in_copy(1, 1).start()
    tail_in.start()
    for i in range(_NFULL):
        bi = i % _NIN
        bo = i % _NOUT
        in_copy(i, bi).wait()
        if i >= _NOUT:
            for c in out_copies(i - _NOUT, bo):
                c.wait()
        o_v[bo] = _dot(wt_ref, ft_v[bi])
        if i + _NIN < _NFULL:
            in_copy(i + _NIN, bi).start()
        for c in out_copies(i, bo):
            c.start()
    tail_in.wait()
    o_t[...] = _dot(wt_ref, ft_t[...])
    for c in tail_outs():
        c.start()
    for i in range(max(0, _NFULL - _NOUT), _NFULL):
        for c in out_copies(i, i % _NOUT):
            c.wait()
    for c in tail_outs():
        c.wait()


def kernel(features, W_fc):
    n, k = features.shape
    h = W_fc.shape[0]
    ft = features.T
    wt = W_fc.T
    out_t = pl.pallas_call(
        _body,
        in_specs=[
            pl.BlockSpec((k, h), lambda: (0, 0)),
            pl.BlockSpec(memory_space=pl.ANY),
        ],
        out_specs=pl.BlockSpec(memory_space=pl.ANY),
        out_shape=jax.ShapeDtypeStruct((h, n), jnp.float32),
        scratch_shapes=[
            pltpu.VMEM((_NIN, k, _BN), jnp.float32),
            pltpu.VMEM((_NOUT, h, _BN), jnp.float32),
            pltpu.VMEM((k, _TAIL), jnp.float32),
            pltpu.VMEM((h, _TAIL), jnp.float32),
            pltpu.SemaphoreType.DMA((_NIN,)),
            pltpu.SemaphoreType.DMA((_NOUT, _S)),
            pltpu.SemaphoreType.DMA,
            pltpu.SemaphoreType.DMA((_S,)),
        ],
    )(wt, ft)
    return out_t.T


# speedup vs baseline: 1.0795x; 1.0795x over previous
import jax
import jax.numpy as jnp
from jax.experimental import pallas as pl
from jax.experimental.pallas import tpu as pltpu

_BLOCK_N = 32768


def _linear_t_body(w_ref, x_ref, o_ref):
    o_ref[...] = jax.lax.dot_general(
        w_ref[...],
        x_ref[...],
        dimension_numbers=(((0,), (0,)), ((), ())),
        preferred_element_type=jnp.float32,
    )


def kernel(features, W_fc):
    n, k = features.shape
    h = W_fc.shape[0]
    ft = features.T
    wt = W_fc.T
    grid = pl.cdiv(n, _BLOCK_N)
    out_t = pl.pallas_call(
        _linear_t_body,
        grid=(grid,),
        in_specs=[
            pl.BlockSpec((k, h), lambda i: (0, 0)),
            pl.BlockSpec((k, _BLOCK_N), lambda i: (0, i)),
        ],
        out_specs=pl.BlockSpec((h, _BLOCK_N), lambda i: (0, i)),
        out_shape=jax.ShapeDtypeStruct((h, n), jnp.float32),
        compiler_params=pltpu.CompilerParams(
            dimension_semantics=("parallel",),
        ),
    )(wt, ft)
    return out_t.T
